# baseline (speedup 1.0000x reference)
import functools
import math

import jax
import jax.numpy as jnp
from jax import lax
from jax.experimental import pallas as pl
from jax.experimental.pallas import tpu as pltpu
from jax.experimental.pallas import tpu_sc as plsc

EMBED_DIM = 64
SCALE = math.sqrt(EMBED_DIM)

NUM_CORES = 2
NUM_SUBCORES = 16
NUM_WORKERS = NUM_CORES * NUM_SUBCORES

GATHER_W = 128
IDX_BLOCK = 200


def _emb_kernel(n_rows):
    rows_per_worker = n_rows // NUM_WORKERS
    wins_per_worker = rows_per_worker // GATHER_W
    assert wins_per_worker % IDX_BLOCK == 0
    mesh = plsc.VectorSubcoreMesh(core_axis_name="c", subcore_axis_name="s")

    @functools.partial(
        pl.kernel,
        mesh=mesh,
        compiler_params=pltpu.CompilerParams(use_tc_tiling_on_sc=False),
        out_type=jax.ShapeDtypeStruct((n_rows, EMBED_DIM), jnp.float32),
        scratch_types=[
            pltpu.VMEM((IDX_BLOCK, GATHER_W), jnp.int32),
            pltpu.VMEM((GATHER_W, EMBED_DIM), jnp.float32),
            pltpu.SemaphoreType.DMA,
        ],
    )
    def k(idx_hbm, table_hbm, out_hbm, idx_v, rows_v, sem):
        wid = lax.axis_index("s") * NUM_CORES + lax.axis_index("c")
        win0 = wid * wins_per_worker

        @pl.loop(0, wins_per_worker // IDX_BLOCK)
        def _(bi):
            pltpu.sync_copy(
                idx_hbm.at[pl.ds(win0 + bi * IDX_BLOCK, IDX_BLOCK)], idx_v
            )

            @pl.loop(0, IDX_BLOCK)
            def _(j):
                pltpu.async_copy(table_hbm.at[idx_v.at[j]], rows_v, sem).wait()

                @pl.loop(0, GATHER_W)
                def _(r):
                    for c in range(0, EMBED_DIM, 16):
                        rows_v[r, pl.ds(c, 16)] = (
                            rows_v[r, pl.ds(c, 16)] * SCALE
                        )

                out_row = (win0 + bi * IDX_BLOCK + j) * GATHER_W
                pltpu.sync_copy(rows_v, out_hbm.at[pl.ds(out_row, GATHER_W)])

    return k


def kernel(x, table):
    b, s = x.shape
    n = b * s
    idx = x.reshape(n // GATHER_W, GATHER_W).astype(jnp.int32)
    out = _emb_kernel(n)(idx, table)
    return out.reshape(b, s, EMBED_DIM)


# speedup vs baseline: 1.2070x; 1.2070x over previous
import functools
import math

import jax
import jax.numpy as jnp
from jax import lax
from jax.experimental import pallas as pl
from jax.experimental.pallas import tpu as pltpu
from jax.experimental.pallas import tpu_sc as plsc

EMBED_DIM = 64
SCALE = math.sqrt(EMBED_DIM)

NUM_CORES = 2
NUM_SUBCORES = 16
NUM_WORKERS = NUM_CORES * NUM_SUBCORES

GATHER_W = 128
NBUF = 4


def _emb_kernel(n_rows):
    rows_per_worker = n_rows // NUM_WORKERS
    n_win = rows_per_worker // GATHER_W
    assert n_win % NBUF == 0 and n_win // NBUF >= 2
    n_outer = n_win // NBUF
    mesh = plsc.VectorSubcoreMesh(core_axis_name="c", subcore_axis_name="s")

    @functools.partial(
        pl.kernel,
        mesh=mesh,
        compiler_params=pltpu.CompilerParams(use_tc_tiling_on_sc=False),
        out_type=jax.ShapeDtypeStruct((n_rows, EMBED_DIM), jnp.float32),
        scratch_types=[
            pltpu.VMEM((n_win, GATHER_W), jnp.int32),
            [pltpu.VMEM((GATHER_W, EMBED_DIM), jnp.float32)] * NBUF,
            [pltpu.VMEM((GATHER_W, EMBED_DIM), jnp.float32)] * NBUF,
            [pltpu.SemaphoreType.DMA] * NBUF,
            [pltpu.SemaphoreType.DMA] * NBUF,
        ],
    )
    def k(idx_hbm, table_hbm, out_hbm, idx_v, inb, outb, gsem, osem):
        wid = lax.axis_index("s") * NUM_CORES + lax.axis_index("c")
        win0 = wid * n_win

        pltpu.sync_copy(idx_hbm.at[pl.ds(win0, n_win)], idx_v)

        def start_gather(j, b):
            pltpu.async_copy(table_hbm.at[idx_v.at[j]], inb[b], gsem[b])

        def wait_gather(b):
            pltpu.make_async_copy(table_hbm.at[idx_v.at[0]], inb[b],
                                  gsem[b]).wait()

        def scale(b):
            @pl.loop(0, GATHER_W)
            def _(r):
                for c in range(0, EMBED_DIM, 16):
                    outb[b][r, pl.ds(c, 16)] = inb[b][r, pl.ds(c, 16)] * SCALE

        def start_out(j, b):
            row = (win0 + j) * GATHER_W
            pltpu.async_copy(outb[b], out_hbm.at[pl.ds(row, GATHER_W)],
                             osem[b])

        def wait_out(b):
            pltpu.make_async_copy(outb[b], out_hbm.at[pl.ds(0, GATHER_W)],
                                  osem[b]).wait()

        for b in range(NBUF):
            start_gather(b, b)

        for b in range(NBUF):
            wait_gather(b)
            scale(b)
            start_gather(NBUF + b, b)
            start_out(b, b)

        @pl.loop(1, n_outer - 1)
        def _(t):
            for b in range(NBUF):
                j = t * NBUF + b
                wait_gather(b)
                wait_out(b)
                scale(b)
                start_gather(j + NBUF, b)
                start_out(j, b)

        for b in range(NBUF):
            wait_gather(b)
            wait_out(b)
            scale(b)
            start_out(n_win - NBUF + b, b)

        for b in range(NBUF):
            wait_out(b)

    return k


def kernel(x, table):
    b, s = x.shape
    n = b * s
    idx = x.reshape(n // GATHER_W, GATHER_W).astype(jnp.int32)
    out = _emb_kernel(n)(idx, table)
    return out.reshape(b, s, EMBED_DIM)
